# Initial kernel scaffold; baseline (speedup 1.0000x reference)
import jax
import jax.numpy as jnp
from jax.experimental import pallas as pl


def kernel(embedding_weights, sharded_sparse_features):
    raise NotImplementedError("write your pallas kernel here")


# speedup vs baseline: 2.7125x; 2.7125x over previous
import functools

import jax
import jax.numpy as jnp
from jax import lax
from jax.experimental import pallas as pl
from jax.experimental.pallas import tpu as pltpu
from jax.experimental.pallas import tpu_sc as plsc

LANES = 16


def _make_sc_kernel(num_rows, T, D, BAGS, L, NC, NS):
    NW = NC * NS
    bags_per_w = BAGS // NW
    CH = 32
    for cand in (128, 96, 64, 32):
        if bags_per_w % cand == 0:
            CH = cand
            break
    assert bags_per_w % CH == 0
    n_chunks = bags_per_w // CH
    IDX_PER_CHUNK = CH * L
    GATHER_W = 128
    n_gathers = IDX_PER_CHUNK // GATHER_W
    assert IDX_PER_CHUNK % GATHER_W == 0
    n_slices = IDX_PER_CHUNK // LANES
    cols_per_row = GATHER_W // LANES

    mesh = plsc.VectorSubcoreMesh(
        core_axis_name="c", subcore_axis_name="s", num_cores=NC, num_subcores=NS
    )

    @functools.partial(
        pl.kernel,
        out_type=jax.ShapeDtypeStruct((BAGS, D), jnp.float32),
        mesh=mesh,
        compiler_params=pltpu.CompilerParams(
            use_tc_tiling_on_sc=False, needs_layout_passes=False
        ),
        scratch_types=[
            pltpu.VMEM((IDX_PER_CHUNK,), jnp.int32),
            pltpu.VMEM((IDX_PER_CHUNK, D), jnp.float32),
            pltpu.VMEM((CH, D), jnp.float32),
            pltpu.SemaphoreType.DMA,
        ],
    )
    def sc_kernel(table_hbm, idx_hbm, out_hbm, fidx_v, rows_v, out_v, sem):
        wid = lax.axis_index("s") * NC + lax.axis_index("c")
        lane = lax.iota(jnp.int32, LANES)

        def chunk_body(ci, carry):
            base_bag = wid * bags_per_w + ci * CH
            base_idx = base_bag * L

            pltpu.sync_copy(idx_hbm.at[pl.ds(base_idx, IDX_PER_CHUNK)], fidx_v)

            def fid_body(s, c):
                sl = pl.ds(s * LANES, LANES)
                raw = fidx_v[sl]
                jg = base_idx + s * LANES + lane
                t = (jg // L) % T
                fidx_v[sl] = raw * T + t
                return c

            lax.fori_loop(0, n_slices, fid_body, 0, unroll=8)

            descs = []
            for j in range(n_gathers):
                descs.append(
                    pltpu.async_copy(
                        table_hbm.at[fidx_v.at[pl.ds(j * GATHER_W, GATHER_W)]],
                        rows_v.at[pl.ds(j * GATHER_W, GATHER_W)],
                        sem,
                    )
                )
            for d in descs:
                d.wait()

            def pool_body(i, c):
                row0 = i * L
                acc_lo = rows_v[row0, pl.ds(0, LANES)]
                acc_hi = rows_v[row0, pl.ds(LANES, LANES)]
                for l in range(1, L):
                    acc_lo = acc_lo + rows_v[row0 + l, pl.ds(0, LANES)]
                    acc_hi = acc_hi + rows_v[row0 + l, pl.ds(LANES, LANES)]
                out_v[i, pl.ds(0, LANES)] = acc_lo
                out_v[i, pl.ds(LANES, LANES)] = acc_hi
                return c

            lax.fori_loop(0, CH, pool_body, 0)

            pltpu.sync_copy(out_v, out_hbm.at[pl.ds(base_bag, CH)])
            return carry

        lax.fori_loop(0, n_chunks, chunk_body, 0)

    return sc_kernel


def kernel(embedding_weights, sharded_sparse_features):
    N, T, D = embedding_weights.shape
    B, T2, L = sharded_sparse_features.shape
    assert T2 == T
    BAGS = B * T

    try:
        info = plsc.get_sparse_core_info()
        NC, NS = info.num_cores, info.num_subcores
    except Exception:
        NC, NS = 2, 16

    table = embedding_weights.reshape(N * T, D)
    idx_flat = sharded_sparse_features.reshape(-1).astype(jnp.int32)

    sc_kernel = _make_sc_kernel(N * T, T, D, BAGS, L, NC, NS)
    out = sc_kernel(table, idx_flat)
    return out.reshape(B, T, D)


# speedup vs baseline: 6.8972x; 2.5427x over previous
import functools

import jax
import jax.numpy as jnp
from jax import lax
from jax.experimental import pallas as pl
from jax.experimental.pallas import tpu as pltpu
from jax.experimental.pallas import tpu_sc as plsc

LANES = 16


def _make_sc_kernel(N, T, D, B, L, NC, NS):
    NW = NC * NS
    assert D == NW, "one subcore per embedding-dim column"
    BBLK = 512
    n_bblk = B // BBLK
    assert B % BBLK == 0

    mesh = plsc.VectorSubcoreMesh(
        core_axis_name="c", subcore_axis_name="s", num_cores=NC, num_subcores=NS
    )

    @functools.partial(
        pl.kernel,
        out_type=jax.ShapeDtypeStruct((T, D, B), jnp.float32),
        mesh=mesh,
        compiler_params=pltpu.CompilerParams(needs_layout_passes=False),
        scratch_types=[
            pltpu.VMEM((N,), jnp.float32),
            pltpu.VMEM((L, BBLK), jnp.int32),
            pltpu.VMEM((B,), jnp.float32),
        ],
    )
    def sc_kernel(w_hbm, idx_hbm, out_hbm, slab_v, idx_v, acc_v):
        d = lax.axis_index("s") * NC + lax.axis_index("c")

        def t_body(t, carry):
            pltpu.sync_copy(w_hbm.at[t, d, :], slab_v)

            def bblk_body(blk, c):
                b0 = blk * BBLK
                pltpu.sync_copy(idx_hbm.at[t, :, pl.ds(b0, BBLK)], idx_v)

                def bv_body(bv, c2):
                    col = bv * LANES
                    acc = plsc.load_gather(slab_v, [idx_v[0, pl.ds(col, LANES)]])
                    for l in range(1, L):
                        acc = acc + plsc.load_gather(
                            slab_v, [idx_v[l, pl.ds(col, LANES)]]
                        )
                    acc_v[pl.ds(b0 + col, LANES)] = acc
                    return c2

                lax.fori_loop(0, BBLK // LANES, bv_body, 0)
                return c

            lax.fori_loop(0, n_bblk, bblk_body, 0)
            pltpu.sync_copy(acc_v, out_hbm.at[t, d, :])
            return carry

        lax.fori_loop(0, T, t_body, 0)

    return sc_kernel


def kernel(embedding_weights, sharded_sparse_features):
    N, T, D = embedding_weights.shape
    B, T2, L = sharded_sparse_features.shape
    assert T2 == T

    try:
        info = plsc.get_sparse_core_info()
        NC, NS = info.num_cores, info.num_subcores
    except Exception:
        NC, NS = 2, 16

    wt = jnp.transpose(embedding_weights, (1, 2, 0))
    it = jnp.transpose(sharded_sparse_features.astype(jnp.int32), (1, 2, 0))

    sc_kernel = _make_sc_kernel(N, T, D, B, L, NC, NS)
    out_t = sc_kernel(wt, it)
    return jnp.transpose(out_t, (2, 0, 1))


# speedup vs baseline: 9.0524x; 1.3125x over previous
import functools

import jax
import jax.numpy as jnp
from jax import lax
from jax.experimental import pallas as pl
from jax.experimental.pallas import tpu as pltpu
from jax.experimental.pallas import tpu_sc as plsc

LANES = 16


def _make_sc_kernel(N, T, D, B, L, NC, NS):
    NW = NC * NS
    assert D == NW, "one subcore per embedding-dim column"
    BBLK = 256
    n_bblk = B // BBLK
    assert B % BBLK == 0 and n_bblk >= 2 and n_bblk % 2 == 0
    assert T >= 2

    mesh = plsc.VectorSubcoreMesh(
        core_axis_name="c", subcore_axis_name="s", num_cores=NC, num_subcores=NS
    )

    @functools.partial(
        pl.kernel,
        out_type=jax.ShapeDtypeStruct((T, D, B), jnp.float32),
        mesh=mesh,
        compiler_params=pltpu.CompilerParams(needs_layout_passes=False),
        scratch_types=[
            pltpu.VMEM((N,), jnp.float32),
            pltpu.VMEM((2, L, BBLK), jnp.int32),
            pltpu.VMEM((2, B), jnp.float32),
            pltpu.SemaphoreType.DMA,
            pltpu.SemaphoreType.DMA((2,)),
            pltpu.SemaphoreType.DMA((2,)),
        ],
    )
    def sc_kernel(
        w_hbm, idx_hbm, out_hbm, slab_v, idx_v, acc_v, sem_slab, sem_idx, sem_acc
    ):
        d = lax.axis_index("s") * NC + lax.axis_index("c")

        def idx_src(t, blk):
            return idx_hbm.at[t, :, pl.ds(blk * BBLK, BBLK)]

        def t_body(t, carry):
            tp = t % 2
            slab_desc = pltpu.async_copy(w_hbm.at[t, d, :], slab_v, sem_slab)
            pltpu.async_copy(idx_src(t, 0), idx_v.at[0], sem_idx.at[0])

            @pl.when(t >= 2)
            def _():
                pltpu.make_async_copy(
                    acc_v.at[tp], out_hbm.at[t - 2, d, :], sem_acc.at[tp]
                ).wait()

            slab_desc.wait()

            def blk_body(blk, c):
                cur = blk % 2
                nxt = (blk + 1) % 2

                @pl.when(blk < n_bblk - 1)
                def _():
                    pltpu.async_copy(
                        idx_src(t, blk + 1), idx_v.at[nxt], sem_idx.at[nxt]
                    )

                pltpu.make_async_copy(
                    idx_src(t, blk), idx_v.at[cur], sem_idx.at[cur]
                ).wait()
                b0 = blk * BBLK

                def bv_body(bv, c2):
                    col = bv * LANES
                    acc = plsc.load_gather(
                        slab_v, [idx_v[cur, 0, pl.ds(col, LANES)]]
                    )
                    for l in range(1, L):
                        acc = acc + plsc.load_gather(
                            slab_v, [idx_v[cur, l, pl.ds(col, LANES)]]
                        )
                    acc_v[tp, pl.ds(b0 + col, LANES)] = acc
                    return c2

                lax.fori_loop(0, BBLK // LANES, bv_body, 0)
                return c

            lax.fori_loop(0, n_bblk, blk_body, 0)
            pltpu.async_copy(acc_v.at[tp], out_hbm.at[t, d, :], sem_acc.at[tp])
            return carry

        lax.fori_loop(0, T, t_body, 0)

        for par in (0, 1):
            t_last = T - 2 + par
            pltpu.make_async_copy(
                acc_v.at[t_last % 2], out_hbm.at[t_last, d, :],
                sem_acc.at[t_last % 2],
            ).wait()

    return sc_kernel


def kernel(embedding_weights, sharded_sparse_features):
    N, T, D = embedding_weights.shape
    B, T2, L = sharded_sparse_features.shape
    assert T2 == T

    try:
        info = plsc.get_sparse_core_info()
        NC, NS = info.num_cores, info.num_subcores
    except Exception:
        NC, NS = 2, 16

    wt = jnp.transpose(embedding_weights, (1, 2, 0))
    it = jnp.transpose(sharded_sparse_features.astype(jnp.int32), (1, 2, 0))

    sc_kernel = _make_sc_kernel(N, T, D, B, L, NC, NS)
    out_t = sc_kernel(wt, it)
    return jnp.transpose(out_t, (2, 0, 1))


# speedup vs baseline: 10.2017x; 1.1270x over previous
import functools

import jax
import jax.numpy as jnp
from jax import lax
from jax.experimental import pallas as pl
from jax.experimental.pallas import tpu as pltpu
from jax.experimental.pallas import tpu_sc as plsc

LANES = 16


def _make_sc_kernel(N, T, D, B, L, NC, NS):
    NW = NC * NS
    assert D == NW, "one subcore per embedding-dim column"
    BBLK = 512
    n_bblk = B // BBLK
    assert B % BBLK == 0 and n_bblk >= 2 and n_bblk % 2 == 0
    assert T >= 2

    mesh = plsc.VectorSubcoreMesh(
        core_axis_name="c", subcore_axis_name="s", num_cores=NC, num_subcores=NS
    )

    @functools.partial(
        pl.kernel,
        out_type=jax.ShapeDtypeStruct((T, D, B), jnp.float32),
        mesh=mesh,
        compiler_params=pltpu.CompilerParams(needs_layout_passes=False),
        scratch_types=[
            pltpu.VMEM((N,), jnp.float32),
            pltpu.VMEM((2, L, BBLK), jnp.int32),
            pltpu.VMEM((B,), jnp.float32),
            pltpu.SemaphoreType.DMA,
            pltpu.SemaphoreType.DMA((2,)),
        ],
    )
    def sc_kernel(
        w_hbm, idx_hbm, out_hbm, slab_v, idx_v, acc_v, sem_slab, sem_idx
    ):
        d = lax.axis_index("s") * NC + lax.axis_index("c")

        def idx_src(t, blk):
            return idx_hbm.at[t, :, pl.ds(blk * BBLK, BBLK)]

        def t_body(t, carry):
            slab_desc = pltpu.async_copy(w_hbm.at[t, d, :], slab_v, sem_slab)
            pltpu.async_copy(idx_src(t, 0), idx_v.at[0], sem_idx.at[0])
            slab_desc.wait()

            def blk_body(blk, c):
                cur = blk % 2
                nxt = (blk + 1) % 2

                @pl.when(blk < n_bblk - 1)
                def _():
                    pltpu.async_copy(
                        idx_src(t, blk + 1), idx_v.at[nxt], sem_idx.at[nxt]
                    )

                pltpu.make_async_copy(
                    idx_src(t, blk), idx_v.at[cur], sem_idx.at[cur]
                ).wait()
                b0 = blk * BBLK

                def bv_body(bv, c2):
                    col = bv * LANES
                    acc0 = plsc.load_gather(
                        slab_v, [idx_v[cur, 0, pl.ds(col, LANES)]]
                    )
                    acc1 = plsc.load_gather(
                        slab_v, [idx_v[cur, 1, pl.ds(col, LANES)]]
                    )
                    for l in range(2, L, 2):
                        acc0 = acc0 + plsc.load_gather(
                            slab_v, [idx_v[cur, l, pl.ds(col, LANES)]]
                        )
                        acc1 = acc1 + plsc.load_gather(
                            slab_v, [idx_v[cur, l + 1, pl.ds(col, LANES)]]
                        )
                    acc_v[pl.ds(b0 + col, LANES)] = acc0 + acc1
                    return c2

                lax.fori_loop(0, BBLK // LANES, bv_body, 0, unroll=2)
                return c

            lax.fori_loop(0, n_bblk, blk_body, 0)
            pltpu.sync_copy(acc_v, out_hbm.at[t, d, :])
            return carry

        lax.fori_loop(0, T, t_body, 0)

    return sc_kernel


def kernel(embedding_weights, sharded_sparse_features):
    N, T, D = embedding_weights.shape
    B, T2, L = sharded_sparse_features.shape
    assert T2 == T

    try:
        info = plsc.get_sparse_core_info()
        NC, NS = info.num_cores, info.num_subcores
    except Exception:
        NC, NS = 2, 16

    wt = jnp.transpose(embedding_weights, (1, 2, 0))
    it = jnp.transpose(sharded_sparse_features.astype(jnp.int32), (1, 2, 0))

    sc_kernel = _make_sc_kernel(N, T, D, B, L, NC, NS)
    out_t = sc_kernel(wt, it)
    return jnp.transpose(out_t, (2, 0, 1))


# speedup vs baseline: 10.5185x; 1.0311x over previous
import functools

import jax
import jax.numpy as jnp
from jax import lax
from jax.experimental import pallas as pl
from jax.experimental.pallas import tpu as pltpu
from jax.experimental.pallas import tpu_sc as plsc

LANES = 16


def _make_sc_kernel(N, T, D, B, L, NC, NS):
    NW = NC * NS
    assert D == NW, "one subcore per embedding-dim column"
    BBLK = 128
    n_bblk = B // BBLK
    assert B % BBLK == 0 and n_bblk >= 2 and n_bblk % 2 == 0
    assert B % (NS * 8) == 0
    SSLC = B // NS
    LPAD = (L + 7) // 8 * 8

    mesh = plsc.VectorSubcoreMesh(
        core_axis_name="c", subcore_axis_name="s", num_cores=NC, num_subcores=NS
    )

    @functools.partial(
        pl.kernel,
        out_type=jax.ShapeDtypeStruct((T, D, B), jnp.float32),
        mesh=mesh,
        compiler_params=pltpu.CompilerParams(needs_layout_passes=False),
        scratch_types=[
            pltpu.VMEM((N,), jnp.float32),
            pltpu.VMEM((2, LPAD, BBLK), jnp.int32),
            pltpu.VMEM((LPAD, SSLC), jnp.int32),
            pltpu.VMEM((B,), jnp.float32),
            pltpu.VMEM_SHARED((2 * LPAD, B), jnp.int32),
            pltpu.SemaphoreType.DMA,
            pltpu.SemaphoreType.DMA((2,)),
            pltpu.SemaphoreType.DMA,
        ],
    )
    def sc_kernel(
        w_hbm, idx_hbm, out_hbm, slab_v, idx_v, stage_v, acc_v, idx_sh,
        sem_slab, sem_idx, sem_stage
    ):
        s = lax.axis_index("s")
        d = s * NC + lax.axis_index("c")
        sb0 = s * SSLC

        def idx_src(t, blk):
            row0 = (t % 2) * LPAD
            return idx_sh.at[pl.ds(row0, LPAD), pl.ds(blk * BBLK, BBLK)]

        def stage_hbm_src(t):
            return idx_hbm.at[t, :, pl.ds(sb0, SSLC)]

        def stage_dst(t):
            row0 = (t % 2) * LPAD
            return idx_sh.at[pl.ds(row0, LPAD), pl.ds(sb0, SSLC)]

        pltpu.sync_copy(stage_hbm_src(0), stage_v)
        pltpu.sync_copy(stage_v, stage_dst(0))
        plsc.subcore_barrier()

        def t_body(t, carry):
            slab_desc = pltpu.async_copy(w_hbm.at[t, d, :], slab_v, sem_slab)
            pltpu.async_copy(idx_src(t, 0), idx_v.at[0], sem_idx.at[0])

            @pl.when(t + 1 < T)
            def _():
                pltpu.async_copy(stage_hbm_src(t + 1), stage_v, sem_stage)

            slab_desc.wait()

            def blk_body(blk, c):
                cur = blk % 2
                nxt = (blk + 1) % 2

                @pl.when(blk < n_bblk - 1)
                def _():
                    pltpu.async_copy(
                        idx_src(t, blk + 1), idx_v.at[nxt], sem_idx.at[nxt]
                    )

                pltpu.make_async_copy(
                    idx_src(t, blk), idx_v.at[cur], sem_idx.at[cur]
                ).wait()
                b0 = blk * BBLK

                def bv_body(bv, c2):
                    col = bv * LANES
                    acc0 = plsc.load_gather(
                        slab_v, [idx_v[cur, 0, pl.ds(col, LANES)]]
                    )
                    acc1 = plsc.load_gather(
                        slab_v, [idx_v[cur, 1, pl.ds(col, LANES)]]
                    )
                    for l in range(2, L, 2):
                        acc0 = acc0 + plsc.load_gather(
                            slab_v, [idx_v[cur, l, pl.ds(col, LANES)]]
                        )
                        acc1 = acc1 + plsc.load_gather(
                            slab_v, [idx_v[cur, l + 1, pl.ds(col, LANES)]]
                        )
                    acc_v[pl.ds(b0 + col, LANES)] = acc0 + acc1
                    return c2

                lax.fori_loop(0, BBLK // LANES, bv_body, 0, unroll=2)
                return c

            lax.fori_loop(0, n_bblk, blk_body, 0)
            pltpu.sync_copy(acc_v, out_hbm.at[t, d, :])

            @pl.when(t + 1 < T)
            def _():
                pltpu.make_async_copy(
                    stage_hbm_src(t + 1), stage_v, sem_stage
                ).wait()
                pltpu.sync_copy(stage_v, stage_dst(t + 1))

            plsc.subcore_barrier()
            return carry

        lax.fori_loop(0, T, t_body, 0)

    return sc_kernel


def kernel(embedding_weights, sharded_sparse_features):
    N, T, D = embedding_weights.shape
    B, T2, L = sharded_sparse_features.shape
    assert T2 == T

    try:
        info = plsc.get_sparse_core_info()
        NC, NS = info.num_cores, info.num_subcores
    except Exception:
        NC, NS = 2, 16

    wt = jnp.transpose(embedding_weights, (1, 2, 0))
    it = jnp.transpose(sharded_sparse_features.astype(jnp.int32), (1, 2, 0))
    lpad = (L + 7) // 8 * 8
    it = jnp.pad(it, ((0, 0), (0, lpad - L), (0, 0)))

    sc_kernel = _make_sc_kernel(N, T, D, B, L, NC, NS)
    out_t = sc_kernel(wt, it)
    return jnp.transpose(out_t, (2, 0, 1))


# speedup vs baseline: 10.6161x; 1.0093x over previous
import functools

import jax
import jax.numpy as jnp
from jax import lax
from jax.experimental import pallas as pl
from jax.experimental.pallas import tpu as pltpu
from jax.experimental.pallas import tpu_sc as plsc

LANES = 16


def _make_sc_kernel(N, T, D, B, L, NC, NS):
    NW = NC * NS
    assert D == NW, "one subcore per embedding-dim column"
    BBLK = 128
    n_bblk = B // BBLK
    assert B % BBLK == 0 and n_bblk >= 2 and n_bblk % 2 == 0
    assert B % (NS * 8) == 0
    SSLC = B // NS
    LPAD = (L + 7) // 8 * 8

    mesh = plsc.VectorSubcoreMesh(
        core_axis_name="c", subcore_axis_name="s", num_cores=NC, num_subcores=NS
    )

    @functools.partial(
        pl.kernel,
        out_type=jax.ShapeDtypeStruct((T, D, B), jnp.float32),
        mesh=mesh,
        compiler_params=pltpu.CompilerParams(needs_layout_passes=False),
        scratch_types=[
            pltpu.VMEM((N,), jnp.float32),
            pltpu.VMEM((2, LPAD, BBLK), jnp.int32),
            pltpu.VMEM((LPAD, SSLC), jnp.int32),
            pltpu.VMEM((B,), jnp.float32),
            pltpu.VMEM_SHARED((2 * LPAD, B), jnp.int32),
            pltpu.SemaphoreType.DMA,
            pltpu.SemaphoreType.DMA((2,)),
            pltpu.SemaphoreType.DMA,
            pltpu.SemaphoreType.DMA,
        ],
    )
    def sc_kernel(
        w_hbm, idx_hbm, out_hbm, slab_v, idx_v, stage_v, acc_v, idx_sh,
        sem_slab, sem_idx, sem_stage, sem_acc
    ):
        s = lax.axis_index("s")
        d = s * NC + lax.axis_index("c")
        sb0 = s * SSLC

        def idx_src(t, blk):
            row0 = (t % 2) * LPAD
            return idx_sh.at[pl.ds(row0, LPAD), pl.ds(blk * BBLK, BBLK)]

        def stage_hbm_src(t):
            return idx_hbm.at[t, :, pl.ds(sb0, SSLC)]

        def stage_dst(t):
            row0 = (t % 2) * LPAD
            return idx_sh.at[pl.ds(row0, LPAD), pl.ds(sb0, SSLC)]

        pltpu.sync_copy(stage_hbm_src(0), stage_v)
        pltpu.sync_copy(stage_v, stage_dst(0))
        plsc.subcore_barrier()

        def t_body(t, carry):
            slab_desc = pltpu.async_copy(w_hbm.at[t, d, :], slab_v, sem_slab)
            pltpu.async_copy(idx_src(t, 0), idx_v.at[0], sem_idx.at[0])

            @pl.when(t + 1 < T)
            def _():
                pltpu.async_copy(stage_hbm_src(t + 1), stage_v, sem_stage)

            slab_desc.wait()

            @pl.when(t >= 1)
            def _():
                pltpu.make_async_copy(
                    acc_v, out_hbm.at[t - 1, d, :], sem_acc
                ).wait()

            def blk_body(blk, c):
                cur = blk % 2
                nxt = (blk + 1) % 2

                @pl.when(blk < n_bblk - 1)
                def _():
                    pltpu.async_copy(
                        idx_src(t, blk + 1), idx_v.at[nxt], sem_idx.at[nxt]
                    )

                pltpu.make_async_copy(
                    idx_src(t, blk), idx_v.at[cur], sem_idx.at[cur]
                ).wait()
                b0 = blk * BBLK

                def bv_body(bv, c2):
                    col = bv * LANES
                    acc0 = plsc.load_gather(
                        slab_v, [idx_v[cur, 0, pl.ds(col, LANES)]]
                    )
                    acc1 = plsc.load_gather(
                        slab_v, [idx_v[cur, 1, pl.ds(col, LANES)]]
                    )
                    for l in range(2, L, 2):
                        acc0 = acc0 + plsc.load_gather(
                            slab_v, [idx_v[cur, l, pl.ds(col, LANES)]]
                        )
                        acc1 = acc1 + plsc.load_gather(
                            slab_v, [idx_v[cur, l + 1, pl.ds(col, LANES)]]
                        )
                    acc_v[pl.ds(b0 + col, LANES)] = acc0 + acc1
                    return c2

                lax.fori_loop(0, BBLK // LANES, bv_body, 0, unroll=2)
                return c

            lax.fori_loop(0, n_bblk, blk_body, 0)
            pltpu.async_copy(acc_v, out_hbm.at[t, d, :], sem_acc)

            @pl.when(t + 1 < T)
            def _():
                pltpu.make_async_copy(
                    stage_hbm_src(t + 1), stage_v, sem_stage
                ).wait()
                pltpu.sync_copy(stage_v, stage_dst(t + 1))

            plsc.subcore_barrier()
            return carry

        lax.fori_loop(0, T, t_body, 0)
        pltpu.make_async_copy(acc_v, out_hbm.at[T - 1, d, :], sem_acc).wait()

    return sc_kernel


def kernel(embedding_weights, sharded_sparse_features):
    N, T, D = embedding_weights.shape
    B, T2, L = sharded_sparse_features.shape
    assert T2 == T

    try:
        info = plsc.get_sparse_core_info()
        NC, NS = info.num_cores, info.num_subcores
    except Exception:
        NC, NS = 2, 16

    wt = jnp.transpose(embedding_weights, (1, 2, 0))
    it = jnp.transpose(sharded_sparse_features.astype(jnp.int32), (1, 2, 0))
    lpad = (L + 7) // 8 * 8
    it = jnp.pad(it, ((0, 0), (0, lpad - L), (0, 0)))

    sc_kernel = _make_sc_kernel(N, T, D, B, L, NC, NS)
    out_t = sc_kernel(wt, it)
    return jnp.transpose(out_t, (2, 0, 1))
